# Initial kernel scaffold; baseline (speedup 1.0000x reference)
import jax
import jax.numpy as jnp
from jax.experimental import pallas as pl


def kernel(x, table):
    raise NotImplementedError("write your pallas kernel here")


# speedup vs baseline: 1.5847x; 1.5847x over previous
import functools

import jax
import jax.numpy as jnp
from jax import lax
from jax.experimental import pallas as pl
from jax.experimental.pallas import tpu as pltpu
from jax.experimental.pallas import tpu_sc as plsc

DIM = 64
NC = 2
NS = 16
NW = NC * NS
CHUNK = 128


@functools.lru_cache(maxsize=None)
def _gather_kernel(B, V):
    b_per_w = B // NW
    n_chunks = b_per_w // CHUNK
    mesh = plsc.VectorSubcoreMesh(core_axis_name="c", subcore_axis_name="s")

    @functools.partial(
        pl.kernel,
        mesh=mesh,
        out_type=jax.ShapeDtypeStruct((B, DIM), jnp.float32),
        scratch_types=[
            pltpu.VMEM((CHUNK,), jnp.int32),
            pltpu.VMEM((CHUNK, DIM), jnp.float32),
            pltpu.SemaphoreType.DMA,
        ],
        compiler_params=pltpu.CompilerParams(use_tc_tiling_on_sc=False),
    )
    def k(idx_hbm, table_hbm, out_hbm, idx_v, rows_v, sem):
        wid = lax.axis_index("s") * NC + lax.axis_index("c")
        base = wid * b_per_w

        def body(i, carry):
            off = base + i * CHUNK
            pltpu.sync_copy(idx_hbm.at[pl.ds(off, CHUNK)], idx_v)
            pltpu.async_copy(table_hbm.at[idx_v], rows_v, sem).wait()
            pltpu.sync_copy(rows_v, out_hbm.at[pl.ds(off, CHUNK)])
            return carry

        lax.fori_loop(0, n_chunks, body, 0)

    return k


@jax.jit
def kernel(x, table):
    B = x.shape[0] * x.shape[1]
    flat = x.reshape(B).astype(jnp.int32)
    out = _gather_kernel(B, table.shape[0])(flat, table)
    return out.reshape(x.shape[0], x.shape[1], DIM)


# speedup vs baseline: 1.8768x; 1.1843x over previous
import functools

import jax
import jax.numpy as jnp
from jax import lax
from jax.experimental import pallas as pl
from jax.experimental.pallas import tpu as pltpu
from jax.experimental.pallas import tpu_sc as plsc

DIM = 64
NC = 2
NS = 16
NW = NC * NS
CHUNK = 128
NBUF = 8
SKEW_I = 4
SKEW_G = 2


@functools.lru_cache(maxsize=None)
def _gather_kernel(B, V):
    b_per_w = B // NW
    n_chunks = b_per_w // CHUNK
    assert n_chunks % NBUF == 0
    mesh = plsc.VectorSubcoreMesh(core_axis_name="c", subcore_axis_name="s")

    @functools.partial(
        pl.kernel,
        mesh=mesh,
        out_type=jax.ShapeDtypeStruct((B, DIM), jnp.float32),
        scratch_types=[
            pltpu.VMEM((NBUF, CHUNK), jnp.int32),
            pltpu.VMEM((NBUF, CHUNK, DIM), jnp.float32),
            pltpu.SemaphoreType.DMA((NBUF,)),
            pltpu.SemaphoreType.DMA((NBUF,)),
            pltpu.SemaphoreType.DMA((NBUF,)),
        ],
        compiler_params=pltpu.CompilerParams(use_tc_tiling_on_sc=False),
    )
    def k(idx_hbm, table_hbm, out_hbm, idx_v, rows_v, sem_i, sem_g, sem_o):
        wid = lax.axis_index("s") * NC + lax.axis_index("c")
        base = wid * b_per_w

        def start_idx(i, slot):
            off = base + i * CHUNK
            pltpu.async_copy(idx_hbm.at[pl.ds(off, CHUNK)], idx_v.at[slot],
                             sem_i.at[slot])

        def wait_idx(i, slot):
            off = base + i * CHUNK
            pltpu.make_async_copy(idx_hbm.at[pl.ds(off, CHUNK)],
                                  idx_v.at[slot], sem_i.at[slot]).wait()

        def start_gather(slot):
            pltpu.async_copy(table_hbm.at[idx_v.at[slot]], rows_v.at[slot],
                             sem_g.at[slot])

        def wait_gather(slot):
            pltpu.make_async_copy(table_hbm.at[idx_v.at[slot]],
                                  rows_v.at[slot], sem_g.at[slot]).wait()

        def start_out(i, slot):
            off = base + i * CHUNK
            pltpu.async_copy(rows_v.at[slot], out_hbm.at[pl.ds(off, CHUNK)],
                             sem_o.at[slot])

        def wait_out(i, slot):
            off = base + i * CHUNK
            pltpu.make_async_copy(rows_v.at[slot],
                                  out_hbm.at[pl.ds(off, CHUNK)],
                                  sem_o.at[slot]).wait()

        for i in range(SKEW_I):
            start_idx(i, i % NBUF)
        for i in range(SKEW_G):
            wait_idx(i, i % NBUF)
            start_gather(i % NBUF)

        def body(g, carry):
            for b in range(NBUF):
                t = g * NBUF + b
                s_i = (b + SKEW_I) % NBUF

                @pl.when(t + SKEW_I < n_chunks)
                def _():
                    start_idx(t + SKEW_I, s_i)

                s_g = (b + SKEW_G) % NBUF

                @pl.when(t + SKEW_G < n_chunks)
                def _():
                    wait_idx(t + SKEW_G, s_g)

                    @pl.when(t + SKEW_G >= NBUF)
                    def _():
                        wait_out(t + SKEW_G - NBUF, s_g)

                    start_gather(s_g)

                wait_gather(b)
                start_out(t, b)
            return carry

        lax.fori_loop(0, n_chunks // NBUF, body, 0)

        for b in range(NBUF):
            wait_out(n_chunks - NBUF + b, b)

    return k


@jax.jit
def kernel(x, table):
    B = x.shape[0] * x.shape[1]
    flat = x.reshape(B).astype(jnp.int32)
    out = _gather_kernel(B, table.shape[0])(flat, table)
    return out.reshape(x.shape[0], x.shape[1], DIM)
